# Initial kernel scaffold; baseline (speedup 1.0000x reference)
import functools

import jax
import jax.numpy as jnp
from jax import lax
from jax.experimental import pallas as pl
from jax.experimental.pallas import tpu as pltpu
from jax.experimental.pallas import tpu_sc as plsc

D_MODEL = 64
N_TOKENS = 16384 * 50

_INFO = plsc.get_sparse_core_info()
NUM_CORES = _INFO.num_cores
NUM_SUBCORES = _INFO.num_subcores
NW = NUM_CORES * NUM_SUBCORES
B_PER_W = N_TOKENS // NW

IDX_W = 128
GATHERS_PER_STEP = 4
CHUNK = IDX_W * GATHERS_PER_STEP
N_STEPS = B_PER_W // CHUNK


def _emb_body(table_hbm, idx_hbm, out_hbm, idx_v, rows_v, gsem):
    wid = lax.axis_index("s") * NUM_CORES + lax.axis_index("c")
    base = wid * B_PER_W

    def step(g, carry):
        off = base + g * CHUNK
        pltpu.sync_copy(idx_hbm.at[pl.ds(off, CHUNK)], idx_v)
        handles = [
            pltpu.async_copy(
                table_hbm.at[idx_v.at[j]],
                rows_v.at[pl.ds(j * IDX_W, IDX_W)],
                gsem,
            )
            for j in range(GATHERS_PER_STEP)
        ]
        for h in handles:
            h.wait()
        pltpu.sync_copy(rows_v, out_hbm.at[pl.ds(off, CHUNK)])
        return carry

    lax.fori_loop(0, N_STEPS, step, 0)


@functools.partial(
    pl.kernel,
    out_type=jax.ShapeDtypeStruct((N_TOKENS, D_MODEL), jnp.float32),
    mesh=plsc.VectorSubcoreMesh(core_axis_name="c", subcore_axis_name="s"),
    scratch_types=[
        pltpu.VMEM((GATHERS_PER_STEP, IDX_W), jnp.int32),
        pltpu.VMEM((CHUNK, D_MODEL), jnp.float32),
        pltpu.SemaphoreType.DMA,
    ],
)
def _emb_kernel(table_hbm, idx_hbm, out_hbm, idx_v, rows_v, gsem):
    _emb_body(table_hbm, idx_hbm, out_hbm, idx_v, rows_v, gsem)


def kernel(x, table):
    idx_flat = x.reshape(-1).astype(jnp.int32)
    out = _emb_kernel(table, idx_flat)
    return out.reshape(x.shape + (D_MODEL,))


# speedup vs baseline: 1.8593x; 1.8593x over previous
import functools

import jax
import jax.numpy as jnp
from jax import lax
from jax.experimental import pallas as pl
from jax.experimental.pallas import tpu as pltpu
from jax.experimental.pallas import tpu_sc as plsc

D_MODEL = 64
N_TOKENS = 16384 * 50

_INFO = plsc.get_sparse_core_info()
NUM_CORES = _INFO.num_cores
NUM_SUBCORES = _INFO.num_subcores
NW = NUM_CORES * NUM_SUBCORES
B_PER_W = N_TOKENS // NW

IDX_W = 128
IDX_ROWS_PER_W = B_PER_W // IDX_W
GATHERS_PER_STEP = 8
CHUNK = IDX_W * GATHERS_PER_STEP
N_STEPS = B_PER_W // CHUNK


def _emb_body(table_hbm, idx_hbm, out_hbm, idx_v, rows_v, gsem):
    wid = lax.axis_index("s") * NUM_CORES + lax.axis_index("c")
    base = wid * B_PER_W
    base_row = wid * IDX_ROWS_PER_W

    pltpu.sync_copy(idx_hbm.at[pl.ds(base_row, IDX_ROWS_PER_W)], idx_v)

    def step(g, carry):
        off = base + g * CHUNK
        handles = [
            pltpu.async_copy(
                table_hbm.at[idx_v.at[g * GATHERS_PER_STEP + j]],
                rows_v.at[pl.ds(j * IDX_W, IDX_W)],
                gsem,
            )
            for j in range(GATHERS_PER_STEP)
        ]
        for h in handles:
            h.wait()
        pltpu.sync_copy(rows_v, out_hbm.at[pl.ds(off, CHUNK)])
        return carry

    lax.fori_loop(0, N_STEPS, step, 0)


@functools.partial(
    pl.kernel,
    out_type=jax.ShapeDtypeStruct((N_TOKENS, D_MODEL), jnp.float32),
    mesh=plsc.VectorSubcoreMesh(core_axis_name="c", subcore_axis_name="s"),
    compiler_params=pltpu.CompilerParams(use_tc_tiling_on_sc=False),
    scratch_types=[
        pltpu.VMEM((IDX_ROWS_PER_W, IDX_W), jnp.int32),
        pltpu.VMEM((CHUNK, D_MODEL), jnp.float32),
        pltpu.SemaphoreType.DMA,
    ],
)
def _emb_kernel(table_hbm, idx_hbm, out_hbm, idx_v, rows_v, gsem):
    _emb_body(table_hbm, idx_hbm, out_hbm, idx_v, rows_v, gsem)


def kernel(x, table):
    idx_flat = x.reshape(-1, IDX_W).astype(jnp.int32)
    out = _emb_kernel(table, idx_flat)
    return out.reshape(x.shape + (D_MODEL,))


# speedup vs baseline: 1.8745x; 1.0082x over previous
import functools

import jax
import jax.numpy as jnp
from jax import lax
from jax.experimental import pallas as pl
from jax.experimental.pallas import tpu as pltpu
from jax.experimental.pallas import tpu_sc as plsc

D_MODEL = 64
N_TOKENS = 16384 * 50

_INFO = plsc.get_sparse_core_info()
NUM_CORES = _INFO.num_cores
NUM_SUBCORES = _INFO.num_subcores
NW = NUM_CORES * NUM_SUBCORES
B_PER_W = N_TOKENS // NW

IDX_W = 128
IDX_ROWS_PER_W = B_PER_W // IDX_W
GATHERS_PER_STEP = 4
CHUNK = IDX_W * GATHERS_PER_STEP
N_STEPS = B_PER_W // CHUNK
N_OUTER = N_STEPS // 2


def _emb_body(table_hbm, idx_hbm, out_hbm, idx_v, rows_v, gsems, osems):
    wid = lax.axis_index("s") * NUM_CORES + lax.axis_index("c")
    base = wid * B_PER_W
    base_row = wid * IDX_ROWS_PER_W

    pltpu.sync_copy(idx_hbm.at[pl.ds(base_row, IDX_ROWS_PER_W)], idx_v)

    def fire_gather(s, b):
        for j in range(GATHERS_PER_STEP):
            pltpu.async_copy(
                table_hbm.at[idx_v.at[s * GATHERS_PER_STEP + j]],
                rows_v.at[b].at[pl.ds(j * IDX_W, IDX_W)],
                gsems.at[b],
            )

    def wait_gather(s, b):
        for j in range(GATHERS_PER_STEP):
            pltpu.make_async_copy(
                table_hbm.at[idx_v.at[s * GATHERS_PER_STEP + j]],
                rows_v.at[b].at[pl.ds(j * IDX_W, IDX_W)],
                gsems.at[b],
            ).wait()

    def fire_out(s, b):
        pltpu.async_copy(
            rows_v.at[b], out_hbm.at[pl.ds(base + s * CHUNK, CHUNK)], osems.at[b]
        )

    def wait_out(s, b):
        pltpu.make_async_copy(
            rows_v.at[b], out_hbm.at[pl.ds(base + s * CHUNK, CHUNK)], osems.at[b]
        ).wait()

    fire_gather(0, 0)

    def outer(g, carry):
        s0 = 2 * g
        pl.when(g > 0)(lambda: wait_out(s0 - 1, 1))
        fire_gather(s0 + 1, 1)
        wait_gather(s0, 0)
        fire_out(s0, 0)
        def prefetch_even():
            wait_out(s0, 0)
            fire_gather(s0 + 2, 0)
        pl.when(g < N_OUTER - 1)(prefetch_even)
        wait_gather(s0 + 1, 1)
        fire_out(s0 + 1, 1)
        return carry

    lax.fori_loop(0, N_OUTER, outer, 0)

    wait_out(N_STEPS - 2, 0)
    wait_out(N_STEPS - 1, 1)


@functools.partial(
    pl.kernel,
    out_type=jax.ShapeDtypeStruct((N_TOKENS, D_MODEL), jnp.float32),
    mesh=plsc.VectorSubcoreMesh(core_axis_name="c", subcore_axis_name="s"),
    compiler_params=pltpu.CompilerParams(use_tc_tiling_on_sc=False),
    scratch_types=[
        pltpu.VMEM((IDX_ROWS_PER_W, IDX_W), jnp.int32),
        pltpu.VMEM((2, CHUNK, D_MODEL), jnp.float32),
        pltpu.SemaphoreType.DMA((2,)),
        pltpu.SemaphoreType.DMA((2,)),
    ],
)
def _emb_kernel(table_hbm, idx_hbm, out_hbm, idx_v, rows_v, gsems, osems):
    _emb_body(table_hbm, idx_hbm, out_hbm, idx_v, rows_v, gsems, osems)


def kernel(x, table):
    idx_flat = x.reshape(-1, IDX_W).astype(jnp.int32)
    out = _emb_kernel(table, idx_flat)
    return out.reshape(x.shape + (D_MODEL,))
